# Initial kernel scaffold; baseline (speedup 1.0000x reference)
import jax
import jax.numpy as jnp
from jax.experimental import pallas as pl


def kernel(inputs, table):
    raise NotImplementedError("write your pallas kernel here")


# speedup vs baseline: 1.4650x; 1.4650x over previous
import jax
import jax.numpy as jnp
from jax import lax
from jax.experimental import pallas as pl
from jax.experimental.pallas import tpu as pltpu
from jax.experimental.pallas import tpu_sc as plsc

_VOCAB = 1000000
_FIELDS = 26
_BATCH = 16384

_info = plsc.get_sparse_core_info()
_NC, _NS, _L = _info.num_cores, _info.num_subcores, _info.num_lanes
_NW = _NC * _NS
_BPW = _BATCH // _NW
_IPW = _BPW * _FIELDS


def _body(table_hbm, idx_hbm, out_hbm, idx_v, vals_v, out_v, sem):
    wid = lax.axis_index("s") * _NC + lax.axis_index("c")
    row0 = wid * _BPW
    i0 = wid * _IPW

    pltpu.sync_copy(idx_hbm.at[pl.ds(i0, _IPW)], idx_v)
    pltpu.async_copy(table_hbm.at[idx_v], vals_v, sem).wait()

    def chunk(c, carry):
        acc = jnp.zeros((_L,), jnp.float32)
        for j in range(_FIELDS):
            acc = acc + vals_v[pl.ds(j * _BPW + c * _L, _L)]
        m = acc * (1.0 / _FIELDS)
        out_v[pl.ds(c * _L, _L)] = 1.0 / (1.0 + jnp.exp(-m))
        return carry

    lax.fori_loop(0, _BPW // _L, chunk, 0)

    pltpu.sync_copy(out_v, out_hbm.at[pl.ds(row0, _BPW)])


def kernel(inputs, table):
    idx = (
        inputs.astype(jnp.int32)
        .reshape(_NW, _BPW, _FIELDS)
        .transpose(0, 2, 1)
        .reshape(-1)
    )
    tab = table.reshape(-1)
    mesh = plsc.VectorSubcoreMesh(core_axis_name="c", subcore_axis_name="s")
    run = pl.kernel(
        _body,
        out_type=jax.ShapeDtypeStruct((_BATCH,), jnp.float32),
        mesh=mesh,
        scratch_types=[
            pltpu.VMEM((_IPW,), jnp.int32),
            pltpu.VMEM((_IPW,), jnp.float32),
            pltpu.VMEM((_BPW,), jnp.float32),
            pltpu.SemaphoreType.DMA,
        ],
    )
    out = run(tab, idx)
    return out.reshape(_BATCH, 1)


# speedup vs baseline: 1.5020x; 1.0253x over previous
import jax
import jax.numpy as jnp
from jax import lax
from jax.experimental import pallas as pl
from jax.experimental.pallas import tpu as pltpu
from jax.experimental.pallas import tpu_sc as plsc

_VOCAB = 1000000
_FIELDS = 26
_BATCH = 16384

_info = plsc.get_sparse_core_info()
_NC, _NS, _L = _info.num_cores, _info.num_subcores, _info.num_lanes
_NW = _NC * _NS
_BPW = _BATCH // _NW
_IPW = _BPW * _FIELDS


def _body(table_hbm, idx_hbm, out_hbm, idx_v, vals_v, out_v, sem, gsem):
    wid = lax.axis_index("s") * _NC + lax.axis_index("c")
    row0 = wid * _BPW

    cps = [
        pltpu.async_copy(
            idx_hbm.at[pl.ds(j * _BATCH + row0, _BPW)],
            idx_v.at[pl.ds(j * _BPW, _BPW)],
            sem,
        )
        for j in range(_FIELDS)
    ]
    for cp in cps:
        cp.wait()

    pltpu.async_copy(table_hbm.at[idx_v], vals_v, gsem).wait()

    def chunk(c, carry):
        acc = jnp.zeros((_L,), jnp.float32)
        for j in range(_FIELDS):
            acc = acc + vals_v[pl.ds(j * _BPW + c * _L, _L)]
        m = acc * (1.0 / _FIELDS)
        out_v[pl.ds(c * _L, _L)] = 1.0 / (1.0 + jnp.exp(-m))
        return carry

    lax.fori_loop(0, _BPW // _L, chunk, 0)

    pltpu.sync_copy(out_v, out_hbm.at[pl.ds(row0, _BPW)])


def kernel(inputs, table):
    idx = lax.optimization_barrier(inputs.T).reshape(-1)
    tab = table.reshape(-1)
    mesh = plsc.VectorSubcoreMesh(core_axis_name="c", subcore_axis_name="s")
    run = pl.kernel(
        _body,
        out_type=jax.ShapeDtypeStruct((_BATCH,), jnp.float32),
        mesh=mesh,
        scratch_types=[
            pltpu.VMEM((_IPW,), jnp.int32),
            pltpu.VMEM((_IPW,), jnp.float32),
            pltpu.VMEM((_BPW,), jnp.float32),
            pltpu.SemaphoreType.DMA,
            pltpu.SemaphoreType.DMA,
        ],
    )
    out = run(tab, idx)
    return out.reshape(_BATCH, 1)


# speedup vs baseline: 1.5131x; 1.0074x over previous
import jax
import jax.numpy as jnp
from jax import lax
from jax.experimental import pallas as pl
from jax.experimental.pallas import tpu as pltpu
from jax.experimental.pallas import tpu_sc as plsc

_VOCAB = 1000000
_FIELDS = 26
_BATCH = 16384

_info = plsc.get_sparse_core_info()
_NC, _NS, _L = _info.num_cores, _info.num_subcores, _info.num_lanes
_NW = _NC * _NS
_BPW = _BATCH // _NW
_IPW = _BPW * _FIELDS


def _body(table_hbm, idx_hbm, out_hbm, idx_v, vals_v, out_v, sem, gsem):
    wid = lax.axis_index("s") * _NC + lax.axis_index("c")
    row0 = wid * _BPW

    cps = [
        pltpu.async_copy(
            idx_hbm.at[pl.ds(j * _BATCH + row0, _BPW)],
            idx_v.at[pl.ds(j * _BPW, _BPW)],
            sem,
        )
        for j in range(_FIELDS)
    ]
    gps = []
    for j in range(_FIELDS):
        cps[j].wait()
        gps.append(
            pltpu.async_copy(
                table_hbm.at[idx_v.at[pl.ds(j * _BPW, _BPW)]],
                vals_v.at[pl.ds(j * _BPW, _BPW)],
                gsem,
            )
        )
    for gp in gps:
        gp.wait()

    def chunk(c, carry):
        acc = jnp.zeros((_L,), jnp.float32)
        for j in range(_FIELDS):
            acc = acc + vals_v[pl.ds(j * _BPW + c * _L, _L)]
        m = acc * (1.0 / _FIELDS)
        out_v[pl.ds(c * _L, _L)] = 1.0 / (1.0 + jnp.exp(-m))
        return carry

    lax.fori_loop(0, _BPW // _L, chunk, 0)

    pltpu.sync_copy(out_v, out_hbm.at[pl.ds(row0, _BPW)])


def kernel(inputs, table):
    idx = lax.optimization_barrier(inputs.T).reshape(-1)
    tab = table.reshape(-1)
    mesh = plsc.VectorSubcoreMesh(core_axis_name="c", subcore_axis_name="s")
    run = pl.kernel(
        _body,
        out_type=jax.ShapeDtypeStruct((_BATCH,), jnp.float32),
        mesh=mesh,
        scratch_types=[
            pltpu.VMEM((_IPW,), jnp.int32),
            pltpu.VMEM((_IPW,), jnp.float32),
            pltpu.VMEM((_BPW,), jnp.float32),
            pltpu.SemaphoreType.DMA,
            pltpu.SemaphoreType.DMA,
        ],
    )
    out = run(tab, idx)
    return out.reshape(_BATCH, 1)


# speedup vs baseline: 3.2369x; 2.1392x over previous
import jax
import jax.numpy as jnp
from jax import lax
from jax.experimental import pallas as pl
from jax.experimental.pallas import tpu as pltpu
from jax.experimental.pallas import tpu_sc as plsc

_VOCAB = 1000000
_FIELDS = 26
_BATCH = 16384

_info = plsc.get_sparse_core_info()
_NC, _NS, _L = _info.num_cores, _info.num_subcores, _info.num_lanes
_NW = _NC * _NS
_BPW = _BATCH // _NW
_IPW = _BPW * _FIELDS


def _body(table_hbm, idx_hbm, out_hbm, idx_v, vals_v, out_v, sem, gsem):
    wid = lax.axis_index("s") * _NC + lax.axis_index("c")
    row0 = wid * _BPW

    cps = [
        pltpu.async_copy(
            idx_hbm.at[pl.ds(j * _BATCH + row0, _BPW)],
            idx_v.at[pl.ds(j * _BPW, _BPW)],
            sem,
        )
        for j in range(_FIELDS)
    ]
    gps = []
    for j in range(_FIELDS):
        cps[j].wait()
        gps.append(
            pltpu.async_copy(
                table_hbm.at[0].at[idx_v.at[pl.ds(j * _BPW, _BPW)]],
                vals_v.at[pl.ds(j * _BPW, _BPW)],
                gsem,
            )
        )
    for gp in gps:
        gp.wait()

    def chunk(c, carry):
        acc = jnp.zeros((_L,), jnp.float32)
        for j in range(_FIELDS):
            acc = acc + vals_v[pl.ds(j * _BPW + c * _L, _L)]
        m = acc * (1.0 / _FIELDS)
        out_v[pl.ds(c * _L, _L)] = 1.0 / (1.0 + jnp.exp(-m))
        return carry

    lax.fori_loop(0, _BPW // _L, chunk, 0)

    pltpu.sync_copy(out_v, out_hbm.at[pl.ds(row0, _BPW)])


def kernel(inputs, table):
    idx = lax.optimization_barrier(inputs.T).reshape(-1)
    tab = lax.optimization_barrier(table.T)
    mesh = plsc.VectorSubcoreMesh(core_axis_name="c", subcore_axis_name="s")
    run = pl.kernel(
        _body,
        out_type=jax.ShapeDtypeStruct((_BATCH,), jnp.float32),
        mesh=mesh,
        scratch_types=[
            pltpu.VMEM((_IPW,), jnp.int32),
            pltpu.VMEM((_IPW,), jnp.float32),
            pltpu.VMEM((_BPW,), jnp.float32),
            pltpu.SemaphoreType.DMA,
            pltpu.SemaphoreType.DMA,
        ],
    )
    out = run(tab, idx)
    return out.reshape(_BATCH, 1)


# speedup vs baseline: 3.2601x; 1.0072x over previous
import jax
import jax.numpy as jnp
from jax import lax
from jax.experimental import pallas as pl
from jax.experimental.pallas import tpu as pltpu
from jax.experimental.pallas import tpu_sc as plsc

_VOCAB = 1000000
_FIELDS = 26
_BATCH = 16384

_info = plsc.get_sparse_core_info()
_NC, _NS, _L = _info.num_cores, _info.num_subcores, _info.num_lanes
_NW = _NC * _NS
_BPW = _BATCH // _NW
_IPW = _BPW * _FIELDS


def _body(table_hbm, idx_hbm, out_hbm, idx_v, vals_v, out_v, sem, gsem):
    wid = lax.axis_index("s") * _NC + lax.axis_index("c")
    row0 = wid * _BPW

    cps = [
        pltpu.async_copy(
            idx_hbm.at[j].at[pl.ds(row0, _BPW)],
            idx_v.at[pl.ds(j * _BPW, _BPW)],
            sem,
        )
        for j in range(_FIELDS)
    ]
    gps = []
    for j in range(_FIELDS):
        cps[j].wait()
        gps.append(
            pltpu.async_copy(
                table_hbm.at[0].at[idx_v.at[pl.ds(j * _BPW, _BPW)]],
                vals_v.at[pl.ds(j * _BPW, _BPW)],
                gsem,
            )
        )
    for gp in gps:
        gp.wait()

    def chunk(c, carry):
        acc = jnp.zeros((_L,), jnp.float32)
        for j in range(_FIELDS):
            acc = acc + vals_v[pl.ds(j * _BPW + c * _L, _L)]
        m = acc * (1.0 / _FIELDS)
        out_v[pl.ds(c * _L, _L)] = 1.0 / (1.0 + jnp.exp(-m))
        return carry

    lax.fori_loop(0, _BPW // _L, chunk, 0)

    pltpu.sync_copy(out_v, out_hbm.at[pl.ds(row0, _BPW)])


def kernel(inputs, table):
    idx = lax.optimization_barrier(inputs.T)
    tab = lax.optimization_barrier(table.T)
    mesh = plsc.VectorSubcoreMesh(core_axis_name="c", subcore_axis_name="s")
    run = pl.kernel(
        _body,
        out_type=jax.ShapeDtypeStruct((_BATCH,), jnp.float32),
        mesh=mesh,
        scratch_types=[
            pltpu.VMEM((_IPW,), jnp.int32),
            pltpu.VMEM((_IPW,), jnp.float32),
            pltpu.VMEM((_BPW,), jnp.float32),
            pltpu.SemaphoreType.DMA,
            pltpu.SemaphoreType.DMA,
        ],
    )
    out = run(tab, idx)
    return out.reshape(_BATCH, 1)
